# baseline (speedup 1.0000x reference)
import functools

import jax
import jax.numpy as jnp
from jax import lax
from jax.experimental import pallas as pl
from jax.experimental.pallas import tpu as pltpu
from jax.experimental.pallas import tpu_sc as plsc

_N = 10000
_E = 320000
_F = 128
_H = 64
_G = 128

_NC = 2
_NS = 16
_NW = _NC * _NS
_CHUNK = 128
_IDX_ROWS = _E // _CHUNK
_ROWS_PER_W = _IDX_ROWS // _NW
_ACC_ROWS = 10240
_NBUF = 8
_NIF = 4

_BLK = 1000
_NBLK = _N // _BLK


def _seg_sum_sc(y, srcm, dstm):
    mesh = plsc.VectorSubcoreMesh(core_axis_name="c", subcore_axis_name="s")

    @functools.partial(
        pl.kernel,
        mesh=mesh,
        compiler_params=pltpu.CompilerParams(use_tc_tiling_on_sc=False),
        out_type=jax.ShapeDtypeStruct((_NC, _ACC_ROWS, _H), jnp.float32),
        scratch_types=[
            pltpu.VMEM((_ROWS_PER_W, _CHUNK), jnp.int32),
            pltpu.VMEM((_ROWS_PER_W, _CHUNK), jnp.int32),
            pltpu.VMEM((_CHUNK,), jnp.int32),
            pltpu.VMEM((_CHUNK,), jnp.int32),
            pltpu.VMEM((_NBUF, _CHUNK, _H), jnp.float32),
            pltpu.VMEM_SHARED((_ACC_ROWS, _H), jnp.float32),
            pltpu.SemaphoreType.DMA((_NBUF,)),
            pltpu.SemaphoreType.DMA((_NBUF,)),
        ],
    )
    def k(y_hbm, srcm_hbm, dstm_hbm, out_hbm, sidx_v, didx_v, esidx_v, edidx_v,
          rows_v, acc_sh, gsem, ssem):
        c = lax.axis_index("c")
        s = lax.axis_index("s")
        wid = c * _NS + s

        pltpu.sync_copy(srcm_hbm.at[pl.ds(wid * _ROWS_PER_W, _ROWS_PER_W)], sidx_v)
        pltpu.sync_copy(dstm_hbm.at[pl.ds(wid * _ROWS_PER_W, _ROWS_PER_W)], didx_v)

        zero16 = jnp.zeros((16,), jnp.float32)
        for r in range(_CHUNK):
            for j in range(_H // 16):
                rows_v[0, r, pl.ds(j * 16, 16)] = zero16
        for b in range(_ACC_ROWS // _NS // _CHUNK):
            pltpu.sync_copy(
                rows_v.at[0],
                acc_sh.at[pl.ds(s * (_ACC_ROWS // _NS) + b * _CHUNK, _CHUNK)],
            )
        plsc.subcore_barrier()

        def gather(ch):
            b = ch % _NBUF
            pltpu.async_copy(y_hbm.at[sidx_v.at[ch]], rows_v.at[b], gsem.at[b])

        def gather_wait(ch):
            b = ch % _NBUF
            pltpu.make_async_copy(
                y_hbm.at[sidx_v.at[ch]], rows_v.at[b], gsem.at[b]
            ).wait()

        def scatter(ch):
            b = ch % _NBUF
            pltpu.async_copy(
                rows_v.at[b], acc_sh.at[didx_v.at[ch]], ssem.at[b], add=True
            )

        def scatter_wait(ch):
            b = ch % _NBUF
            pltpu.make_async_copy(
                rows_v.at[b], acc_sh.at[didx_v.at[ch]], ssem.at[b]
            ).wait()

        for ch in range(_NIF):
            gather(ch)
        for ch in range(_ROWS_PER_W):
            nxt = ch + _NIF
            if nxt < _ROWS_PER_W:
                if nxt >= _NBUF:
                    scatter_wait(nxt - _NBUF)
                gather(nxt)
            gather_wait(ch)
            scatter(ch)
        for ch in range(_ROWS_PER_W - _NBUF, _ROWS_PER_W):
            scatter_wait(ch)

        @pl.when(wid < _IDX_ROWS - _NW * _ROWS_PER_W)
        def _():
            row = _NW * _ROWS_PER_W + wid
            pltpu.sync_copy(srcm_hbm.at[row], esidx_v)
            pltpu.async_copy(y_hbm.at[esidx_v], rows_v.at[0], gsem.at[0]).wait()
            pltpu.sync_copy(dstm_hbm.at[row], edidx_v)
            pltpu.sync_copy(rows_v.at[0], acc_sh.at[edidx_v], add=True)

        plsc.subcore_barrier()

        rows_out = _ACC_ROWS // _NS
        pltpu.sync_copy(
            acc_sh.at[pl.ds(s * rows_out, rows_out)],
            out_hbm.at[c, pl.ds(s * rows_out, rows_out)],
        )

    return k(y, srcm, dstm)


def _mm_body(x_ref, w_ref, o_ref):
    o_ref[...] = jnp.dot(x_ref[...], w_ref[...], preferred_element_type=jnp.float32)


def _mm_head(x, w):
    f = x.shape[1]
    return pl.pallas_call(
        _mm_body,
        grid=(_NBLK,),
        in_specs=[
            pl.BlockSpec((_BLK, f), lambda i: (i, 0)),
            pl.BlockSpec((f, _H), lambda i: (0, 0)),
        ],
        out_specs=pl.BlockSpec((_BLK, _H), lambda i: (i, 0)),
        out_shape=jax.ShapeDtypeStruct((_N, _H), jnp.float32),
    )(x, w)


def _tail_head_body(y_ref, p_ref, ba_ref, wb_ref, bb_ref, wn_ref, o_ref):
    t = jnp.maximum(y_ref[...] + p_ref[0] + p_ref[1] + ba_ref[...], 0.0)
    z = jnp.dot(t, wb_ref[...], preferred_element_type=jnp.float32) + bb_ref[...]
    o_ref[...] = jnp.dot(
        jnp.maximum(z, 0.0), wn_ref[...], preferred_element_type=jnp.float32
    )


def _tail_head(y, p, ba, wb, bb, wn):
    return pl.pallas_call(
        _tail_head_body,
        grid=(_NBLK,),
        in_specs=[
            pl.BlockSpec((_BLK, _H), lambda i: (i, 0)),
            pl.BlockSpec((_NC, _BLK, _H), lambda i: (0, i, 0)),
            pl.BlockSpec((1, _H), lambda i: (0, 0)),
            pl.BlockSpec((_H, _H), lambda i: (0, 0)),
            pl.BlockSpec((1, _H), lambda i: (0, 0)),
            pl.BlockSpec((_H, _H), lambda i: (0, 0)),
        ],
        out_specs=pl.BlockSpec((_BLK, _H), lambda i: (i, 0)),
        out_shape=jax.ShapeDtypeStruct((_N, _H), jnp.float32),
    )(y, p, ba, wb, bb, wn)


def _final_body(y_ref, p_ref, ba_ref, wb_ref, bb_ref, wf_ref, bf_ref, batch_ref,
                o_ref, acc_ref):
    i = pl.program_id(0)
    t = jnp.maximum(y_ref[...] + p_ref[0] + p_ref[1] + ba_ref[...], 0.0)
    z = jnp.dot(t, wb_ref[...], preferred_element_type=jnp.float32) + bb_ref[...]
    v = jnp.dot(z, wf_ref[...], preferred_element_type=jnp.float32)
    b2 = batch_ref[0]
    seg = lax.broadcasted_iota(jnp.int32, (_G, _BLK), 0)
    oh = (seg == b2).astype(jnp.float32)
    vv = jnp.concatenate([v, jnp.ones_like(v)], axis=1)
    contrib = jnp.dot(oh, vv, preferred_element_type=jnp.float32)

    @pl.when(i == 0)
    def _():
        acc_ref[...] = jnp.zeros_like(acc_ref)

    acc_ref[...] += contrib

    @pl.when(i == _NBLK - 1)
    def _():
        sums = acc_ref[:, 0:1]
        cnt = acc_ref[:, 1:2]
        o_ref[...] = sums / jnp.maximum(cnt, 1.0) + bf_ref[...]


def _final(y, p, ba, wb, bb, wf, bf, batch3):
    return pl.pallas_call(
        _final_body,
        grid=(_NBLK,),
        in_specs=[
            pl.BlockSpec((_BLK, _H), lambda i: (i, 0)),
            pl.BlockSpec((_NC, _BLK, _H), lambda i: (0, i, 0)),
            pl.BlockSpec((1, _H), lambda i: (0, 0)),
            pl.BlockSpec((_H, _H), lambda i: (0, 0)),
            pl.BlockSpec((1, _H), lambda i: (0, 0)),
            pl.BlockSpec((_H, 1), lambda i: (0, 0)),
            pl.BlockSpec((1, 1), lambda i: (0, 0)),
            pl.BlockSpec((1, 1, _BLK), lambda i: (i, 0, 0)),
        ],
        out_specs=pl.BlockSpec((_G, 1), lambda i: (0, 0)),
        out_shape=jax.ShapeDtypeStruct((_G, 1), jnp.float32),
        scratch_shapes=[pltpu.VMEM((_G, 2), jnp.float32)],
    )(y, p, ba, wb, bb, wf, bf, batch3)


def kernel(x, edge_index, batch, W1a, b1a, W1b, b1b, W2a, b2a, W2b, b2b,
           W3a, b3a, W3b, b3b, Wf, bf):
    srcm = edge_index[0].reshape(_IDX_ROWS, _CHUNK)
    dstm = edge_index[1].reshape(_IDX_ROWS, _CHUNK)
    batch3 = batch.reshape(_NBLK, 1, _BLK)

    y1 = _mm_head(x, W1a)
    p1 = _seg_sum_sc(y1, srcm, dstm)
    y2 = _tail_head(y1, p1, b1a.reshape(1, _H), W1b, b1b.reshape(1, _H), W2a)
    p2 = _seg_sum_sc(y2, srcm, dstm)
    y3 = _tail_head(y2, p2, b2a.reshape(1, _H), W2b, b2b.reshape(1, _H), W3a)
    p3 = _seg_sum_sc(y3, srcm, dstm)
    return _final(y3, p3, b3a.reshape(1, _H), W3b, b3b.reshape(1, _H),
                  Wf, bf.reshape(1, 1), batch3)


# speedup vs baseline: 1.2704x; 1.2704x over previous
import functools

import jax
import jax.numpy as jnp
from jax import lax
from jax.experimental import pallas as pl
from jax.experimental.pallas import tpu as pltpu
from jax.experimental.pallas import tpu_sc as plsc

_N = 10000
_E = 320000
_F = 128
_H = 64
_G = 128
_NH = _N // 2

_NC = 2
_NS = 16
_NW = _NC * _NS
_CHUNK = 128
_IDX_ROWS = _E // _CHUNK
_ROWS_PER_W = _IDX_ROWS // _NW
_ACC_ROWS = 10240
_NBUF = 8
_NIF = 4

_BLK = 1000
_NBLK = _NH // _BLK


def _seg_sum_sc(y, ei3):
    mesh = plsc.VectorSubcoreMesh(core_axis_name="c", subcore_axis_name="s")

    @functools.partial(
        pl.kernel,
        mesh=mesh,
        compiler_params=pltpu.CompilerParams(use_tc_tiling_on_sc=False),
        out_type=jax.ShapeDtypeStruct((_NC, _ACC_ROWS, _H), jnp.float32),
        scratch_types=[
            pltpu.VMEM((_ROWS_PER_W, _CHUNK), jnp.int32),
            pltpu.VMEM((_ROWS_PER_W, _CHUNK), jnp.int32),
            pltpu.VMEM((_CHUNK,), jnp.int32),
            pltpu.VMEM((_CHUNK,), jnp.int32),
            pltpu.VMEM((_NBUF, _CHUNK, _H), jnp.float32),
            pltpu.VMEM_SHARED((_ACC_ROWS, _H), jnp.float32),
            pltpu.SemaphoreType.DMA((_NBUF,)),
            pltpu.SemaphoreType.DMA((_NBUF,)),
        ],
    )
    def k(y_hbm, ei_hbm, out_hbm, sidx_v, didx_v, esidx_v, edidx_v,
          rows_v, acc_sh, gsem, ssem):
        c = lax.axis_index("c")
        s = lax.axis_index("s")
        wid = c * _NS + s

        pltpu.sync_copy(ei_hbm.at[0, pl.ds(wid * _ROWS_PER_W, _ROWS_PER_W)],
                        sidx_v)
        pltpu.sync_copy(ei_hbm.at[1, pl.ds(wid * _ROWS_PER_W, _ROWS_PER_W)],
                        didx_v)

        zero16 = jnp.zeros((16,), jnp.float32)
        for r in range(_CHUNK):
            for j in range(_H // 16):
                rows_v[0, r, pl.ds(j * 16, 16)] = zero16
        for b in range(_ACC_ROWS // _NS // _CHUNK):
            pltpu.sync_copy(
                rows_v.at[0],
                acc_sh.at[pl.ds(s * (_ACC_ROWS // _NS) + b * _CHUNK, _CHUNK)],
            )
        plsc.subcore_barrier()

        def gather(ch):
            b = ch % _NBUF
            pltpu.async_copy(y_hbm.at[sidx_v.at[ch]], rows_v.at[b], gsem.at[b])

        def gather_wait(ch):
            b = ch % _NBUF
            pltpu.make_async_copy(
                y_hbm.at[sidx_v.at[ch]], rows_v.at[b], gsem.at[b]
            ).wait()

        def scatter(ch):
            b = ch % _NBUF
            pltpu.async_copy(
                rows_v.at[b], acc_sh.at[didx_v.at[ch]], ssem.at[b], add=True
            )

        def scatter_wait(ch):
            b = ch % _NBUF
            pltpu.make_async_copy(
                rows_v.at[b], acc_sh.at[didx_v.at[ch]], ssem.at[b]
            ).wait()

        for ch in range(_NIF):
            gather(ch)
        for ch in range(_ROWS_PER_W):
            nxt = ch + _NIF
            if nxt < _ROWS_PER_W:
                if nxt >= _NBUF:
                    scatter_wait(nxt - _NBUF)
                gather(nxt)
            gather_wait(ch)
            scatter(ch)
        for ch in range(_ROWS_PER_W - _NBUF, _ROWS_PER_W):
            scatter_wait(ch)

        @pl.when(wid < _IDX_ROWS - _NW * _ROWS_PER_W)
        def _():
            row = _NW * _ROWS_PER_W + wid
            pltpu.sync_copy(ei_hbm.at[0, row], esidx_v)
            pltpu.async_copy(y_hbm.at[esidx_v], rows_v.at[0], gsem.at[0]).wait()
            pltpu.sync_copy(ei_hbm.at[1, row], edidx_v)
            pltpu.sync_copy(rows_v.at[0], acc_sh.at[edidx_v], add=True)

        plsc.subcore_barrier()

        rows_out = _ACC_ROWS // _NS
        pltpu.sync_copy(
            acc_sh.at[pl.ds(s * rows_out, rows_out)],
            out_hbm.at[c, pl.ds(s * rows_out, rows_out)],
        )

    return k(y, ei3)


def _head_body(xa_ref, xb_ref, w_ref, o_ref):
    ya = jnp.dot(xa_ref[...], w_ref[...], preferred_element_type=jnp.float32)
    yb = jnp.dot(xb_ref[...], w_ref[...], preferred_element_type=jnp.float32)
    o_ref[...] = jnp.concatenate([ya, yb], axis=1)


def _mm_head(x, w):
    return pl.pallas_call(
        _head_body,
        grid=(_NBLK,),
        in_specs=[
            pl.BlockSpec((_BLK, _F), lambda i: (i, 0)),
            pl.BlockSpec((_BLK, _F), lambda i: (i + _NBLK, 0)),
            pl.BlockSpec((_F, _H), lambda i: (0, 0)),
        ],
        out_specs=pl.BlockSpec((_BLK, 2 * _H), lambda i: (i, 0)),
        out_shape=jax.ShapeDtypeStruct((_NH, 2 * _H), jnp.float32),
    )(x, x, w)


def _tail_head_body(y_ref, p_ref, ba_ref, wb_ref, bb_ref, wn_ref, o_ref):
    t = jnp.maximum(y_ref[...] + p_ref[0] + p_ref[1] + ba_ref[...], 0.0)
    z = jnp.dot(t, wb_ref[...], preferred_element_type=jnp.float32) + bb_ref[...]
    o_ref[...] = jnp.dot(
        jnp.maximum(z, 0.0), wn_ref[...], preferred_element_type=jnp.float32
    )


def _tail_head(y, pv, ba2, wbd, bb2, wnd):
    return pl.pallas_call(
        _tail_head_body,
        grid=(_NBLK,),
        in_specs=[
            pl.BlockSpec((_BLK, 2 * _H), lambda i: (i, 0)),
            pl.BlockSpec((_NC, _BLK, 2 * _H), lambda i: (0, i, 0)),
            pl.BlockSpec((1, 2 * _H), lambda i: (0, 0)),
            pl.BlockSpec((2 * _H, 2 * _H), lambda i: (0, 0)),
            pl.BlockSpec((1, 2 * _H), lambda i: (0, 0)),
            pl.BlockSpec((2 * _H, 2 * _H), lambda i: (0, 0)),
        ],
        out_specs=pl.BlockSpec((_BLK, 2 * _H), lambda i: (i, 0)),
        out_shape=jax.ShapeDtypeStruct((_NH, 2 * _H), jnp.float32),
    )(y, pv, ba2, wbd, bb2, wnd)


def _final_body(y_ref, p_ref, ba_ref, wb_ref, bb_ref, wfd_ref, bf_ref,
                bl_ref, br_ref, o_ref, acc_ref):
    i = pl.program_id(0)
    t = jnp.maximum(y_ref[...] + p_ref[0] + p_ref[1] + ba_ref[...], 0.0)
    z = jnp.dot(t, wb_ref[...], preferred_element_type=jnp.float32) + bb_ref[...]
    v2 = jnp.dot(z, wfd_ref[...], preferred_element_type=jnp.float32)
    ones = jnp.ones((_BLK, 1), jnp.float32)
    seg = lax.broadcasted_iota(jnp.int32, (_G, _BLK), 0)
    ohl = (seg == bl_ref[0]).astype(jnp.float32)
    ohr = (seg == br_ref[0]).astype(jnp.float32)
    contrib = jnp.dot(
        ohl, jnp.concatenate([v2[:, 0:1], ones], axis=1),
        preferred_element_type=jnp.float32,
    ) + jnp.dot(
        ohr, jnp.concatenate([v2[:, 1:2], ones], axis=1),
        preferred_element_type=jnp.float32,
    )

    @pl.when(i == 0)
    def _():
        acc_ref[...] = jnp.zeros_like(acc_ref)

    acc_ref[...] += contrib

    @pl.when(i == _NBLK - 1)
    def _():
        sums = acc_ref[:, 0:1]
        cnt = acc_ref[:, 1:2]
        o_ref[...] = sums / jnp.maximum(cnt, 1.0) + bf_ref[...]


def _final(y, pv, ba2, wbd, bb2, wfd, bf, bl3, br3):
    return pl.pallas_call(
        _final_body,
        grid=(_NBLK,),
        in_specs=[
            pl.BlockSpec((_BLK, 2 * _H), lambda i: (i, 0)),
            pl.BlockSpec((_NC, _BLK, 2 * _H), lambda i: (0, i, 0)),
            pl.BlockSpec((1, 2 * _H), lambda i: (0, 0)),
            pl.BlockSpec((2 * _H, 2 * _H), lambda i: (0, 0)),
            pl.BlockSpec((1, 2 * _H), lambda i: (0, 0)),
            pl.BlockSpec((2 * _H, 2), lambda i: (0, 0)),
            pl.BlockSpec((1, 1), lambda i: (0, 0)),
            pl.BlockSpec((1, 1, _BLK), lambda i: (i, 0, 0)),
            pl.BlockSpec((1, 1, _BLK), lambda i: (i, 0, 0)),
        ],
        out_specs=pl.BlockSpec((_G, 1), lambda i: (0, 0)),
        out_shape=jax.ShapeDtypeStruct((_G, 1), jnp.float32),
        scratch_shapes=[pltpu.VMEM((_G, 2), jnp.float32)],
    )(y, pv, ba2, wbd, bb2, wfd, bf, bl3, br3)


def _bd(w):
    z = jnp.zeros_like(w)
    return jnp.concatenate(
        [jnp.concatenate([w, z], axis=1), jnp.concatenate([z, w], axis=1)],
        axis=0,
    )


def _b2(b):
    return jnp.concatenate([b, b]).reshape(1, -1)


def kernel(x, edge_index, batch, W1a, b1a, W1b, b1b, W2a, b2a, W2b, b2b,
           W3a, b3a, W3b, b3b, Wf, bf):
    eip = (edge_index % _NH) * 2 + edge_index // _NH
    ei3 = eip.reshape(2, _IDX_ROWS, _CHUNK)
    bl3 = batch[:_NH].reshape(_NBLK, 1, _BLK)
    br3 = batch[_NH:].reshape(_NBLK, 1, _BLK)

    w1bd, w2ad, w2bd, w3ad, w3bd = map(_bd, (W1b, W2a, W2b, W3a, W3b))
    wfd = _bd(Wf)
    b1a2, b1b2, b2a2, b2b2, b3a2, b3b2 = map(
        _b2, (b1a, b1b, b2a, b2b, b3a, b3b))

    y1 = _mm_head(x, W1a)
    p1 = _seg_sum_sc(y1.reshape(_N, _H), ei3).reshape(_NC, _ACC_ROWS // 2, _F)
    y2 = _tail_head(y1, p1, b1a2, w1bd, b1b2, w2ad)
    p2 = _seg_sum_sc(y2.reshape(_N, _H), ei3).reshape(_NC, _ACC_ROWS // 2, _F)
    y3 = _tail_head(y2, p2, b2a2, w2bd, b2b2, w3ad)
    p3 = _seg_sum_sc(y3.reshape(_N, _H), ei3).reshape(_NC, _ACC_ROWS // 2, _F)
    return _final(y3, p3, b3a2, w3bd, b3b2, wfd, bf.reshape(1, 1), bl3, br3)


# speedup vs baseline: 1.3034x; 1.0260x over previous
import functools

import jax
import jax.numpy as jnp
from jax import lax
from jax.experimental import pallas as pl
from jax.experimental.pallas import tpu as pltpu
from jax.experimental.pallas import tpu_sc as plsc

_N = 10000
_E = 320000
_F = 128
_H = 64
_G = 128
_NH = _N // 2

_NC = 2
_NS = 16
_NW = _NC * _NS
_CHUNK = 128
_IDX_ROWS = _E // _CHUNK
_ROWS_PER_W = _IDX_ROWS // _NW
_ACC_ROWS = 10240
_NBUF = 8
_NIF = 4

_BLK = 1000
_NBLK = _NH // _BLK


def _seg_sum_sc(y, ei3):
    mesh = plsc.VectorSubcoreMesh(core_axis_name="c", subcore_axis_name="s")

    @functools.partial(
        pl.kernel,
        mesh=mesh,
        compiler_params=pltpu.CompilerParams(use_tc_tiling_on_sc=False),
        out_type=jax.ShapeDtypeStruct((_NC, _ACC_ROWS, _H), jnp.float32),
        scratch_types=[
            pltpu.VMEM((_ROWS_PER_W, _CHUNK), jnp.int32),
            pltpu.VMEM((_ROWS_PER_W, _CHUNK), jnp.int32),
            pltpu.VMEM((_CHUNK,), jnp.int32),
            pltpu.VMEM((_CHUNK,), jnp.int32),
            pltpu.VMEM((_NBUF, _CHUNK, _H), jnp.float32),
            pltpu.VMEM_SHARED((_ACC_ROWS, _H), jnp.float32),
            pltpu.SemaphoreType.DMA((_NBUF,)),
            pltpu.SemaphoreType.DMA((_NBUF,)),
        ],
    )
    def k(y_hbm, ei_hbm, out_hbm, sidx_v, didx_v, esidx_v, edidx_v,
          rows_v, acc_sh, gsem, ssem):
        c = lax.axis_index("c")
        s = lax.axis_index("s")
        wid = c * _NS + s

        pltpu.sync_copy(ei_hbm.at[0, pl.ds(wid * _ROWS_PER_W, _ROWS_PER_W)],
                        sidx_v)
        pltpu.sync_copy(ei_hbm.at[1, pl.ds(wid * _ROWS_PER_W, _ROWS_PER_W)],
                        didx_v)

        zero16 = jnp.zeros((16,), jnp.float32)
        for r in range(_CHUNK):
            for j in range(_H // 16):
                rows_v[0, r, pl.ds(j * 16, 16)] = zero16
        for b in range(_ACC_ROWS // _NS // _CHUNK):
            pltpu.sync_copy(
                rows_v.at[0],
                acc_sh.at[pl.ds(s * (_ACC_ROWS // _NS) + b * _CHUNK, _CHUNK)],
            )
        plsc.subcore_barrier()

        def gather(ch):
            b = ch % _NBUF
            pltpu.async_copy(y_hbm.at[sidx_v.at[ch]], rows_v.at[b], gsem.at[b])

        def gather_wait(ch):
            b = ch % _NBUF
            pltpu.make_async_copy(
                y_hbm.at[sidx_v.at[ch]], rows_v.at[b], gsem.at[b]
            ).wait()

        def scatter(ch):
            b = ch % _NBUF
            pltpu.async_copy(
                rows_v.at[b], acc_sh.at[didx_v.at[ch]], ssem.at[b], add=True
            )

        def scatter_wait(ch):
            b = ch % _NBUF
            pltpu.make_async_copy(
                rows_v.at[b], acc_sh.at[didx_v.at[ch]], ssem.at[b]
            ).wait()

        for ch in range(_NIF):
            gather(ch)
        for ch in range(_ROWS_PER_W):
            nxt = ch + _NIF
            if nxt < _ROWS_PER_W:
                if nxt >= _NBUF:
                    scatter_wait(nxt - _NBUF)
                gather(nxt)
            gather_wait(ch)
            scatter(ch)
        for ch in range(_ROWS_PER_W - _NBUF, _ROWS_PER_W):
            scatter_wait(ch)

        @pl.when(wid < _IDX_ROWS - _NW * _ROWS_PER_W)
        def _():
            row = _NW * _ROWS_PER_W + wid
            pltpu.sync_copy(ei_hbm.at[0, row], esidx_v)
            pltpu.async_copy(y_hbm.at[esidx_v], rows_v.at[0], gsem.at[0]).wait()
            pltpu.sync_copy(ei_hbm.at[1, row], edidx_v)
            pltpu.sync_copy(rows_v.at[0], acc_sh.at[edidx_v], add=True)

        plsc.subcore_barrier()

        rows_out = _ACC_ROWS // _NS
        pltpu.sync_copy(
            acc_sh.at[pl.ds(s * rows_out, rows_out)],
            out_hbm.at[c, pl.ds(s * rows_out, rows_out)],
        )

    return k(y, ei3)


def _head_body(x_ref, w_ref, o_ref):
    o_ref[...] = jnp.dot(x_ref[...], w_ref[...],
                         preferred_element_type=jnp.float32)


def _mm_head(xp, wd):
    return pl.pallas_call(
        _head_body,
        grid=(_NBLK,),
        in_specs=[
            pl.BlockSpec((_BLK, 2 * _F), lambda i: (i, 0)),
            pl.BlockSpec((2 * _F, 2 * _H), lambda i: (0, 0)),
        ],
        out_specs=pl.BlockSpec((_BLK, 2 * _H), lambda i: (i, 0)),
        out_shape=jax.ShapeDtypeStruct((_NH, 2 * _H), jnp.float32),
    )(xp, wd)


def _tail_head_body(y_ref, p_ref, ba_ref, wb_ref, bb_ref, wn_ref, o_ref):
    t = jnp.maximum(y_ref[...] + p_ref[0] + p_ref[1] + ba_ref[...], 0.0)
    z = jnp.dot(t, wb_ref[...], preferred_element_type=jnp.float32) + bb_ref[...]
    o_ref[...] = jnp.dot(
        jnp.maximum(z, 0.0), wn_ref[...], preferred_element_type=jnp.float32
    )


def _tail_head(y, pv, ba2, wbd, bb2, wnd):
    return pl.pallas_call(
        _tail_head_body,
        grid=(_NBLK,),
        in_specs=[
            pl.BlockSpec((_BLK, 2 * _H), lambda i: (i, 0)),
            pl.BlockSpec((_NC, _BLK, 2 * _H), lambda i: (0, i, 0)),
            pl.BlockSpec((1, 2 * _H), lambda i: (0, 0)),
            pl.BlockSpec((2 * _H, 2 * _H), lambda i: (0, 0)),
            pl.BlockSpec((1, 2 * _H), lambda i: (0, 0)),
            pl.BlockSpec((2 * _H, 2 * _H), lambda i: (0, 0)),
        ],
        out_specs=pl.BlockSpec((_BLK, 2 * _H), lambda i: (i, 0)),
        out_shape=jax.ShapeDtypeStruct((_NH, 2 * _H), jnp.float32),
    )(y, pv, ba2, wbd, bb2, wnd)


def _final_body(y_ref, p_ref, ba_ref, wb_ref, bb_ref, wfd_ref, bf_ref,
                bl_ref, br_ref, o_ref, acc_ref):
    i = pl.program_id(0)
    t = jnp.maximum(y_ref[...] + p_ref[0] + p_ref[1] + ba_ref[...], 0.0)
    z = jnp.dot(t, wb_ref[...], preferred_element_type=jnp.float32) + bb_ref[...]
    v2 = jnp.dot(z, wfd_ref[...], preferred_element_type=jnp.float32)
    ones = jnp.ones((_BLK, 1), jnp.float32)
    seg = lax.broadcasted_iota(jnp.int32, (_G, _BLK), 0)
    ohl = (seg == bl_ref[0]).astype(jnp.float32)
    ohr = (seg == br_ref[0]).astype(jnp.float32)
    contrib = jnp.dot(
        ohl, jnp.concatenate([v2[:, 0:1], ones], axis=1),
        preferred_element_type=jnp.float32,
    ) + jnp.dot(
        ohr, jnp.concatenate([v2[:, 1:2], ones], axis=1),
        preferred_element_type=jnp.float32,
    )

    @pl.when(i == 0)
    def _():
        acc_ref[...] = jnp.zeros_like(acc_ref)

    acc_ref[...] += contrib

    @pl.when(i == _NBLK - 1)
    def _():
        sums = acc_ref[:, 0:1]
        cnt = acc_ref[:, 1:2]
        o_ref[...] = sums / jnp.maximum(cnt, 1.0) + bf_ref[...]


def _final(y, pv, ba2, wbd, bb2, wfd, bf, bl3, br3):
    return pl.pallas_call(
        _final_body,
        grid=(_NBLK,),
        in_specs=[
            pl.BlockSpec((_BLK, 2 * _H), lambda i: (i, 0)),
            pl.BlockSpec((_NC, _BLK, 2 * _H), lambda i: (0, i, 0)),
            pl.BlockSpec((1, 2 * _H), lambda i: (0, 0)),
            pl.BlockSpec((2 * _H, 2 * _H), lambda i: (0, 0)),
            pl.BlockSpec((1, 2 * _H), lambda i: (0, 0)),
            pl.BlockSpec((2 * _H, 2), lambda i: (0, 0)),
            pl.BlockSpec((1, 1), lambda i: (0, 0)),
            pl.BlockSpec((1, 1, _BLK), lambda i: (i, 0, 0)),
            pl.BlockSpec((1, 1, _BLK), lambda i: (i, 0, 0)),
        ],
        out_specs=pl.BlockSpec((_G, 1), lambda i: (0, 0)),
        out_shape=jax.ShapeDtypeStruct((_G, 1), jnp.float32),
        scratch_shapes=[pltpu.VMEM((_G, 2), jnp.float32)],
    )(y, pv, ba2, wbd, bb2, wfd, bf, bl3, br3)


def _bd(w):
    z = jnp.zeros_like(w)
    return jnp.concatenate(
        [jnp.concatenate([w, z], axis=1), jnp.concatenate([z, w], axis=1)],
        axis=0,
    )


def _b2(b):
    return jnp.concatenate([b, b]).reshape(1, -1)


def kernel(x, edge_index, batch, W1a, b1a, W1b, b1b, W2a, b2a, W2b, b2b,
           W3a, b3a, W3b, b3b, Wf, bf):
    ei3 = edge_index.reshape(2, _IDX_ROWS, _CHUNK)
    bl3 = batch[0::2].reshape(_NBLK, 1, _BLK)
    br3 = batch[1::2].reshape(_NBLK, 1, _BLK)

    xp = x.reshape(_NH, 2 * _F)
    w1ad, w1bd, w2ad, w2bd, w3ad, w3bd = map(
        _bd, (W1a, W1b, W2a, W2b, W3a, W3b))
    wfd = _bd(Wf)
    b1a2, b1b2, b2a2, b2b2, b3a2, b3b2 = map(
        _b2, (b1a, b1b, b2a, b2b, b3a, b3b))

    y1 = _mm_head(xp, w1ad)
    p1 = _seg_sum_sc(y1.reshape(_N, _H), ei3).reshape(_NC, _ACC_ROWS // 2, _F)
    y2 = _tail_head(y1, p1, b1a2, w1bd, b1b2, w2ad)
    p2 = _seg_sum_sc(y2.reshape(_N, _H), ei3).reshape(_NC, _ACC_ROWS // 2, _F)
    y3 = _tail_head(y2, p2, b2a2, w2bd, b2b2, w3ad)
    p3 = _seg_sum_sc(y3.reshape(_N, _H), ei3).reshape(_NC, _ACC_ROWS // 2, _F)
    return _final(y3, p3, b3a2, w3bd, b3b2, wfd, bf.reshape(1, 1), bl3, br3)
